# Initial kernel scaffold; baseline (speedup 1.0000x reference)
import jax
import jax.numpy as jnp
from jax.experimental import pallas as pl


def kernel(hyperedge_seq, epoch, usr_emb, W1, b1, W2, b2, W_ih, W_hh, b_ih, b_hh, W_out, b_out):
    raise NotImplementedError("write your pallas kernel here")


# speedup vs baseline: 13.5114x; 13.5114x over previous
import functools

import jax
import jax.numpy as jnp
from jax import lax
from jax.experimental import pallas as pl
from jax.experimental.pallas import tpu as pltpu
from jax.experimental.pallas import tpu_sc as plsc

N = 10000
T = 8
E = 160000
D = 32
L = 16
NS = 16
NC = 2
EPT = E // NS
K = 1000
NCH = EPT // K
RPT = N // NS
TPC = T // NC
_U = 5


def _fill_rows(ref, nrows, ncols, val):
    v = jnp.full((L,), val, jnp.float32)

    def body(j, _):
        r = j * _U
        for u in range(_U):
            for col in range(ncols // L):
                ref[r + u, col * L:(col + 1) * L] = v
        return 0

    lax.fori_loop(0, nrows // _U, body, 0)


def _offset_idx(idx_ref, off):
    def body(j, _):
        sl = pl.ds(j * L, L)
        idx_ref[sl] = idx_ref[sl] + off
        return 0

    lax.fori_loop(0, K // L, body, 0)


def _stage1_body(y0_hbm, idx_hbm, b1_hbm,
                 x1_hbm, binv_hbm, dinv_hbm, m_hbm,
                 acc, cntB, cntD,
                 idx_n, idx_e, rows, ones, fsl, csl, bv):
    c = lax.axis_index("c")
    s = lax.axis_index("s")
    r0 = s * RPT

    _fill_rows(ones, K, L, 1.0)
    pltpu.sync_copy(b1_hbm, bv)

    for k4 in range(TPC):
        t = c * TPC + k4
        tbase = t * (2 * E)

        _fill_rows(fsl, RPT, D, 0.0)
        _fill_rows(csl, RPT, L, 0.0)
        pltpu.sync_copy(fsl, acc.at[pl.ds(r0, RPT)])
        pltpu.sync_copy(csl, cntB.at[pl.ds(r0, RPT)])
        pltpu.sync_copy(csl, cntD.at[pl.ds(r0, RPT)])
        plsc.subcore_barrier()

        for ch in range(NCH):
            off = s * EPT + ch * K
            pltpu.sync_copy(idx_hbm.at[pl.ds(tbase + off, K)], idx_n)
            pltpu.sync_copy(idx_hbm.at[pl.ds(tbase + E + off, K)], idx_e)
            pltpu.sync_copy(y0_hbm.at[idx_n], rows)
            pltpu.sync_copy(rows, acc.at[idx_e], add=True)
            pltpu.sync_copy(ones, cntB.at[idx_e], add=True)
        plsc.subcore_barrier()

        pltpu.sync_copy(acc.at[pl.ds(r0, RPT)], fsl)
        pltpu.sync_copy(cntB.at[pl.ds(r0, RPT)], csl)

        def s1(j, _):
            for u in range(_U):
                r = j * _U + u
                cv = csl[r]
                inv = jnp.where(cv > 0.0, 1.0 / cv, 0.0)
                csl[r] = inv
                fsl[r, 0:L] = fsl[r, 0:L] * inv
                fsl[r, L:D] = fsl[r, L:D] * inv
            return 0

        lax.fori_loop(0, RPT // _U, s1, 0)
        pltpu.sync_copy(fsl, m_hbm.at[t].at[pl.ds(r0, RPT)])
        pltpu.sync_copy(csl, binv_hbm.at[pl.ds(t * N + r0, RPT)])
        _fill_rows(fsl, RPT, D, 0.0)
        pltpu.sync_copy(fsl, acc.at[pl.ds(r0, RPT)])
        plsc.subcore_barrier()

        for ch in range(NCH):
            off = s * EPT + ch * K
            pltpu.sync_copy(idx_hbm.at[pl.ds(tbase + off, K)], idx_n)
            pltpu.sync_copy(idx_hbm.at[pl.ds(tbase + E + off, K)], idx_e)
            pltpu.sync_copy(m_hbm.at[t].at[idx_e], rows)
            pltpu.sync_copy(rows, acc.at[idx_n], add=True)
            pltpu.sync_copy(ones, cntD.at[idx_n], add=True)
        plsc.subcore_barrier()

        pltpu.sync_copy(acc.at[pl.ds(r0, RPT)], fsl)
        pltpu.sync_copy(cntD.at[pl.ds(r0, RPT)], csl)
        b0 = bv[0:L]
        b1v = bv[L:D]

        def s2(j, _):
            for u in range(_U):
                r = j * _U + u
                cv = csl[r]
                inv = jnp.where(cv > 0.0, 1.0 / cv, 0.0)
                csl[r] = inv
                fsl[r, 0:L] = jnp.maximum(fsl[r, 0:L] * inv + b0, 0.0)
                fsl[r, L:D] = jnp.maximum(fsl[r, L:D] * inv + b1v, 0.0)
            return 0

        lax.fori_loop(0, RPT // _U, s2, 0)
        pltpu.sync_copy(fsl, x1_hbm.at[pl.ds(t * N + r0, RPT)])
        pltpu.sync_copy(csl, dinv_hbm.at[pl.ds(t * N + r0, RPT)])


def _stage3_body(z_hbm, idx_hbm, b2_hbm, binv_hbm, dinv_hbm,
                 x2_hbm, m_hbm,
                 acc,
                 idx_n, idx_e, rows, fsl, csl, bv):
    c = lax.axis_index("c")
    s = lax.axis_index("s")
    r0 = s * RPT

    pltpu.sync_copy(b2_hbm, bv)

    for k4 in range(TPC):
        t = c * TPC + k4
        tbase = t * (2 * E)

        _fill_rows(fsl, RPT, D, 0.0)
        pltpu.sync_copy(fsl, acc.at[pl.ds(r0, RPT)])
        plsc.subcore_barrier()

        for ch in range(NCH):
            off = s * EPT + ch * K
            pltpu.sync_copy(idx_hbm.at[pl.ds(tbase + off, K)], idx_n)
            pltpu.sync_copy(idx_hbm.at[pl.ds(tbase + E + off, K)], idx_e)
            pltpu.sync_copy(z_hbm.at[t].at[idx_n], rows)
            pltpu.sync_copy(rows, acc.at[idx_e], add=True)
        plsc.subcore_barrier()

        pltpu.sync_copy(acc.at[pl.ds(r0, RPT)], fsl)
        pltpu.sync_copy(binv_hbm.at[pl.ds(t * N + r0, RPT)], csl)

        def s1(j, _):
            for u in range(_U):
                r = j * _U + u
                inv = csl[r]
                fsl[r, 0:L] = fsl[r, 0:L] * inv
                fsl[r, L:D] = fsl[r, L:D] * inv
            return 0

        lax.fori_loop(0, RPT // _U, s1, 0)
        pltpu.sync_copy(fsl, m_hbm.at[t].at[pl.ds(r0, RPT)])
        _fill_rows(fsl, RPT, D, 0.0)
        pltpu.sync_copy(fsl, acc.at[pl.ds(r0, RPT)])
        plsc.subcore_barrier()

        for ch in range(NCH):
            off = s * EPT + ch * K
            pltpu.sync_copy(idx_hbm.at[pl.ds(tbase + off, K)], idx_n)
            pltpu.sync_copy(idx_hbm.at[pl.ds(tbase + E + off, K)], idx_e)
            pltpu.sync_copy(m_hbm.at[t].at[idx_e], rows)
            pltpu.sync_copy(rows, acc.at[idx_n], add=True)
        plsc.subcore_barrier()

        pltpu.sync_copy(acc.at[pl.ds(r0, RPT)], fsl)
        pltpu.sync_copy(dinv_hbm.at[pl.ds(t * N + r0, RPT)], csl)
        b0 = bv[0:L]
        b1v = bv[L:D]

        def s2(j, _):
            for u in range(_U):
                r = j * _U + u
                inv = csl[r]
                fsl[r, 0:L] = jnp.maximum(fsl[r, 0:L] * inv + b0, 0.0)
                fsl[r, L:D] = jnp.maximum(fsl[r, L:D] * inv + b1v, 0.0)
            return 0

        lax.fori_loop(0, RPT // _U, s2, 0)
        pltpu.sync_copy(fsl, x2_hbm.at[pl.ds(t * N + r0, RPT)])


_SC_MESH = plsc.VectorSubcoreMesh(core_axis_name="c", subcore_axis_name="s")

_stage1 = pl.kernel(
    _stage1_body,
    out_type=[
        jax.ShapeDtypeStruct((T * N, D), jnp.float32),
        jax.ShapeDtypeStruct((T * N, L), jnp.float32),
        jax.ShapeDtypeStruct((T * N, L), jnp.float32),
        jax.ShapeDtypeStruct((T, N, D), jnp.float32),
    ],
    mesh=_SC_MESH,
    scratch_types=[
        pltpu.VMEM_SHARED((N, D), jnp.float32),
        pltpu.VMEM_SHARED((N, L), jnp.float32),
        pltpu.VMEM_SHARED((N, L), jnp.float32),
        pltpu.VMEM((K,), jnp.int32),
        pltpu.VMEM((K,), jnp.int32),
        pltpu.VMEM((K, D), jnp.float32),
        pltpu.VMEM((K, L), jnp.float32),
        pltpu.VMEM((RPT, D), jnp.float32),
        pltpu.VMEM((RPT, L), jnp.float32),
        pltpu.VMEM((D,), jnp.float32),
    ],
    compiler_params=pltpu.CompilerParams(use_tc_tiling_on_sc=False),
)

_stage3 = pl.kernel(
    _stage3_body,
    out_type=[
        jax.ShapeDtypeStruct((T * N, D), jnp.float32),
        jax.ShapeDtypeStruct((T, N, D), jnp.float32),
    ],
    mesh=_SC_MESH,
    scratch_types=[
        pltpu.VMEM_SHARED((N, D), jnp.float32),
        pltpu.VMEM((K,), jnp.int32),
        pltpu.VMEM((K,), jnp.int32),
        pltpu.VMEM((K, D), jnp.float32),
        pltpu.VMEM((RPT, D), jnp.float32),
        pltpu.VMEM((RPT, L), jnp.float32),
        pltpu.VMEM((D,), jnp.float32),
    ],
    compiler_params=pltpu.CompilerParams(use_tc_tiling_on_sc=False),
)


def _mm_body(x_ref, w_ref, o_ref):
    o_ref[...] = jnp.dot(x_ref[...], w_ref[...],
                         preferred_element_type=jnp.float32)


def _matmul(x, w):
    m = x.shape[0]
    grid = m // N
    return pl.pallas_call(
        _mm_body,
        grid=(grid,),
        in_specs=[
            pl.BlockSpec((N, D), lambda i: (i, 0)),
            pl.BlockSpec((D, D), lambda i: (0, 0)),
        ],
        out_specs=pl.BlockSpec((N, D), lambda i: (i, 0)),
        out_shape=jax.ShapeDtypeStruct((m, D), jnp.float32),
    )(x, w)


_LSTM_BN = 2000


def _lstm_body(seq_ref, wih_ref, whh_ref, bih_ref, bhh_ref,
               wout_ref, bout_ref, o_ref):
    xall = seq_ref[...]
    wih = wih_ref[...]
    whh = whh_ref[...]
    b = bih_ref[...] + bhh_ref[...]
    h = jnp.zeros((_LSTM_BN, 16), jnp.float32)
    c = jnp.zeros((_LSTM_BN, 16), jnp.float32)
    for t in range(T):
        g = jnp.dot(xall[t], wih, preferred_element_type=jnp.float32) \
            + jnp.dot(h, whh, preferred_element_type=jnp.float32) + b
        i = jax.nn.sigmoid(g[:, 0:16])
        f = jax.nn.sigmoid(g[:, 16:32])
        gg = jnp.tanh(g[:, 32:48])
        o = jax.nn.sigmoid(g[:, 48:64])
        c = f * c + i * gg
        h = o * jnp.tanh(c)
    o_ref[...] = jnp.dot(jnp.maximum(h, 0.0), wout_ref[...],
                         preferred_element_type=jnp.float32) + bout_ref[...]


def _lstm_head(x2, wih_t, whh_t, bih, bhh, wout_t, bout):
    grid = N // _LSTM_BN
    return pl.pallas_call(
        _lstm_body,
        grid=(grid,),
        in_specs=[
            pl.BlockSpec((T, _LSTM_BN, D), lambda i: (0, i, 0)),
            pl.BlockSpec((D, 64), lambda i: (0, 0)),
            pl.BlockSpec((16, 64), lambda i: (0, 0)),
            pl.BlockSpec((1, 64), lambda i: (0, 0)),
            pl.BlockSpec((1, 64), lambda i: (0, 0)),
            pl.BlockSpec((16, 2), lambda i: (0, 0)),
            pl.BlockSpec((1, 2), lambda i: (0, 0)),
        ],
        out_specs=pl.BlockSpec((_LSTM_BN, 2), lambda i: (i, 0)),
        out_shape=jax.ShapeDtypeStruct((N, 2), jnp.float32),
    )(x2, wih_t, whh_t, bih, bhh, wout_t, bout)


def kernel(hyperedge_seq, epoch, usr_emb, W1, b1, W2, b2,
           W_ih, W_hh, b_ih, b_hh, W_out, b_out):
    del epoch
    idx_flat = hyperedge_seq.reshape(-1)

    y0 = _matmul(usr_emb, W1)
    x1, binv, dinv, _ = _stage1(y0, idx_flat, b1)
    z = _matmul(x1, W2)
    x2, _ = _stage3(z.reshape(T, N, D), idx_flat, b2, binv, dinv)

    logits = _lstm_head(
        x2.reshape(T, N, D),
        W_ih.T, W_hh.T,
        b_ih.reshape(1, 64), b_hh.reshape(1, 64),
        W_out.T, b_out.reshape(1, 2),
    )
    return logits


# speedup vs baseline: 15.5108x; 1.1480x over previous
import functools

import jax
import jax.numpy as jnp
from jax import lax
from jax.experimental import pallas as pl
from jax.experimental.pallas import tpu as pltpu
from jax.experimental.pallas import tpu_sc as plsc

N = 10000
T = 8
E = 160000
D = 32
DA = 48
L = 16
NS = 16
NC = 2
EPT = E // NS
K1 = 400
NCH1 = EPT // K1
K3 = 500
NCH3 = EPT // K3
RPT = N // NS
TPC = T // NC
_U = 5


def _fill_rows(ref, nrows, ncols, val):
    v = jnp.full((L,), val, jnp.float32)

    def body(j, _):
        r = j * _U
        for u in range(_U):
            for col in range(ncols // L):
                ref[r + u, col * L:(col + 1) * L] = v
        return 0

    lax.fori_loop(0, nrows // _U, body, 0)


def _pass(table_view, idxg_ref, idxs_ref, rows2, acc, nch, sem_g, sem_s):
    prev = [None, None]
    for ch in range(nch):
        b = ch & 1
        if prev[b] is not None:
            prev[b].wait()
        g = pltpu.async_copy(table_view.at[idxg_ref.at[ch]], rows2[b], sem_g)
        g.wait()
        prev[b] = pltpu.async_copy(rows2[b], acc.at[idxs_ref.at[ch]], sem_s,
                                   add=True)
    for p in prev:
        if p is not None:
            p.wait()


def _stage1_body(y0_hbm, idx_hbm, b1_hbm,
                 x1_hbm, binv_hbm, dinv_hbm, m_hbm,
                 acc, idx_n2, idx_e2, rows_a, rows_b, fsl, csl, bv,
                 sem_i, sem_g, sem_s):
    c = lax.axis_index("c")
    s = lax.axis_index("s")
    r0 = s * RPT
    rows2 = [rows_a, rows_b]

    pltpu.sync_copy(b1_hbm, bv)

    for k4 in range(TPC):
        t = c * TPC + k4

        _fill_rows(fsl, RPT, DA, 0.0)
        pltpu.sync_copy(fsl, acc.at[pl.ds(r0, RPT)])
        di = pltpu.async_copy(idx_hbm.at[t].at[0].at[s], idx_n2, sem_i)
        de = pltpu.async_copy(idx_hbm.at[t].at[1].at[s], idx_e2, sem_i)
        di.wait()
        de.wait()
        plsc.subcore_barrier()

        _pass(y0_hbm, idx_n2, idx_e2, rows2, acc, NCH1, sem_g, sem_s)
        plsc.subcore_barrier()

        pltpu.sync_copy(acc.at[pl.ds(r0, RPT)], fsl)
        onev = jnp.full((L,), 1.0, jnp.float32)

        def s1(j, _):
            for u in range(_U):
                r = j * _U + u
                cv = fsl[r, D:DA]
                inv = jnp.where(cv > 0.0, 1.0 / cv, 0.0)
                csl[r] = inv
                fsl[r, 0:L] = fsl[r, 0:L] * inv
                fsl[r, L:D] = fsl[r, L:D] * inv
                fsl[r, D:DA] = onev
            return 0

        lax.fori_loop(0, RPT // _U, s1, 0)
        pltpu.sync_copy(fsl, m_hbm.at[t].at[pl.ds(r0, RPT)])
        pltpu.sync_copy(csl, binv_hbm.at[pl.ds(t * N + r0, RPT)])
        _fill_rows(fsl, RPT, DA, 0.0)
        pltpu.sync_copy(fsl, acc.at[pl.ds(r0, RPT)])
        plsc.subcore_barrier()

        _pass(m_hbm.at[t], idx_e2, idx_n2, rows2, acc, NCH1, sem_g, sem_s)
        plsc.subcore_barrier()

        pltpu.sync_copy(acc.at[pl.ds(r0, RPT)], fsl)
        b0 = bv[0:L]
        b1v = bv[L:D]
        zv = jnp.zeros((L,), jnp.float32)

        def s2(j, _):
            for u in range(_U):
                r = j * _U + u
                cv = fsl[r, D:DA]
                inv = jnp.where(cv > 0.0, 1.0 / cv, 0.0)
                csl[r] = inv
                fsl[r, 0:L] = jnp.maximum(fsl[r, 0:L] * inv + b0, 0.0)
                fsl[r, L:D] = jnp.maximum(fsl[r, L:D] * inv + b1v, 0.0)
                fsl[r, D:DA] = zv
            return 0

        lax.fori_loop(0, RPT // _U, s2, 0)
        pltpu.sync_copy(fsl, x1_hbm.at[pl.ds(t * N + r0, RPT)])
        pltpu.sync_copy(csl, dinv_hbm.at[pl.ds(t * N + r0, RPT)])


def _stage3_body(z_hbm, idx_hbm, b2_hbm, binv_hbm, dinv_hbm,
                 x2_hbm, m_hbm,
                 acc, idx_n2, idx_e2, rows_a, rows_b, fsl, csl, bv,
                 sem_i, sem_g, sem_s):
    c = lax.axis_index("c")
    s = lax.axis_index("s")
    r0 = s * RPT
    rows2 = [rows_a, rows_b]

    pltpu.sync_copy(b2_hbm, bv)

    for k4 in range(TPC):
        t = c * TPC + k4

        _fill_rows(fsl, RPT, D, 0.0)
        pltpu.sync_copy(fsl, acc.at[pl.ds(r0, RPT)])
        di = pltpu.async_copy(idx_hbm.at[t].at[0].at[s], idx_n2, sem_i)
        de = pltpu.async_copy(idx_hbm.at[t].at[1].at[s], idx_e2, sem_i)
        di.wait()
        de.wait()
        plsc.subcore_barrier()

        _pass(z_hbm.at[t], idx_n2, idx_e2, rows2, acc, NCH3, sem_g, sem_s)
        plsc.subcore_barrier()

        pltpu.sync_copy(acc.at[pl.ds(r0, RPT)], fsl)
        pltpu.sync_copy(binv_hbm.at[pl.ds(t * N + r0, RPT)], csl)

        def s1(j, _):
            for u in range(_U):
                r = j * _U + u
                inv = csl[r]
                fsl[r, 0:L] = fsl[r, 0:L] * inv
                fsl[r, L:D] = fsl[r, L:D] * inv
            return 0

        lax.fori_loop(0, RPT // _U, s1, 0)
        pltpu.sync_copy(fsl, m_hbm.at[t].at[pl.ds(r0, RPT)])
        _fill_rows(fsl, RPT, D, 0.0)
        pltpu.sync_copy(fsl, acc.at[pl.ds(r0, RPT)])
        plsc.subcore_barrier()

        _pass(m_hbm.at[t], idx_e2, idx_n2, rows2, acc, NCH3, sem_g, sem_s)
        plsc.subcore_barrier()

        pltpu.sync_copy(acc.at[pl.ds(r0, RPT)], fsl)
        pltpu.sync_copy(dinv_hbm.at[pl.ds(t * N + r0, RPT)], csl)
        b0 = bv[0:L]
        b1v = bv[L:D]

        def s2(j, _):
            for u in range(_U):
                r = j * _U + u
                inv = csl[r]
                fsl[r, 0:L] = jnp.maximum(fsl[r, 0:L] * inv + b0, 0.0)
                fsl[r, L:D] = jnp.maximum(fsl[r, L:D] * inv + b1v, 0.0)
            return 0

        lax.fori_loop(0, RPT // _U, s2, 0)
        pltpu.sync_copy(fsl, x2_hbm.at[pl.ds(t * N + r0, RPT)])


_SC_MESH = plsc.VectorSubcoreMesh(core_axis_name="c", subcore_axis_name="s")

_stage1 = pl.kernel(
    _stage1_body,
    out_type=[
        jax.ShapeDtypeStruct((T * N, DA), jnp.float32),
        jax.ShapeDtypeStruct((T * N, L), jnp.float32),
        jax.ShapeDtypeStruct((T * N, L), jnp.float32),
        jax.ShapeDtypeStruct((T, N, DA), jnp.float32),
    ],
    mesh=_SC_MESH,
    scratch_types=[
        pltpu.VMEM_SHARED((N, DA), jnp.float32),
        pltpu.VMEM((NCH1, K1), jnp.int32),
        pltpu.VMEM((NCH1, K1), jnp.int32),
        pltpu.VMEM((K1, DA), jnp.float32),
        pltpu.VMEM((K1, DA), jnp.float32),
        pltpu.VMEM((RPT, DA), jnp.float32),
        pltpu.VMEM((RPT, L), jnp.float32),
        pltpu.VMEM((D,), jnp.float32),
        pltpu.SemaphoreType.DMA,
        pltpu.SemaphoreType.DMA,
        pltpu.SemaphoreType.DMA,
    ],
    compiler_params=pltpu.CompilerParams(use_tc_tiling_on_sc=False),
)

_stage3 = pl.kernel(
    _stage3_body,
    out_type=[
        jax.ShapeDtypeStruct((T * N, D), jnp.float32),
        jax.ShapeDtypeStruct((T, N, D), jnp.float32),
    ],
    mesh=_SC_MESH,
    scratch_types=[
        pltpu.VMEM_SHARED((N, D), jnp.float32),
        pltpu.VMEM((NCH3, K3), jnp.int32),
        pltpu.VMEM((NCH3, K3), jnp.int32),
        pltpu.VMEM((K3, D), jnp.float32),
        pltpu.VMEM((K3, D), jnp.float32),
        pltpu.VMEM((RPT, D), jnp.float32),
        pltpu.VMEM((RPT, L), jnp.float32),
        pltpu.VMEM((D,), jnp.float32),
        pltpu.SemaphoreType.DMA,
        pltpu.SemaphoreType.DMA,
        pltpu.SemaphoreType.DMA,
    ],
    compiler_params=pltpu.CompilerParams(use_tc_tiling_on_sc=False),
)


def _mm_aug_body(x_ref, w_ref, o_ref):
    xw = jnp.dot(x_ref[...], w_ref[...], preferred_element_type=jnp.float32)
    o_ref[...] = jnp.concatenate(
        [xw, jnp.ones((xw.shape[0], L), jnp.float32)], axis=1)


def _mm_aug(x, w):
    return pl.pallas_call(
        _mm_aug_body,
        grid=(1,),
        in_specs=[
            pl.BlockSpec((N, D), lambda i: (i, 0)),
            pl.BlockSpec((D, D), lambda i: (0, 0)),
        ],
        out_specs=pl.BlockSpec((N, DA), lambda i: (i, 0)),
        out_shape=jax.ShapeDtypeStruct((N, DA), jnp.float32),
    )(x, w)


def _mm_body(x_ref, w_ref, o_ref):
    o_ref[...] = jnp.dot(x_ref[...], w_ref[...],
                         preferred_element_type=jnp.float32)


def _mm_wide(x, w):
    m = x.shape[0]
    grid = m // N
    return pl.pallas_call(
        _mm_body,
        grid=(grid,),
        in_specs=[
            pl.BlockSpec((N, DA), lambda i: (i, 0)),
            pl.BlockSpec((DA, D), lambda i: (0, 0)),
        ],
        out_specs=pl.BlockSpec((N, D), lambda i: (i, 0)),
        out_shape=jax.ShapeDtypeStruct((m, D), jnp.float32),
    )(x, w)


_LSTM_BN = 2000


def _lstm_body(seq_ref, wih_ref, whh_ref, bih_ref, bhh_ref,
               wout_ref, bout_ref, o_ref):
    xall = seq_ref[...]
    wih = wih_ref[...]
    whh = whh_ref[...]
    b = bih_ref[...] + bhh_ref[...]
    h = jnp.zeros((_LSTM_BN, 16), jnp.float32)
    c = jnp.zeros((_LSTM_BN, 16), jnp.float32)
    for t in range(T):
        g = jnp.dot(xall[t], wih, preferred_element_type=jnp.float32) \
            + jnp.dot(h, whh, preferred_element_type=jnp.float32) + b
        i = jax.nn.sigmoid(g[:, 0:16])
        f = jax.nn.sigmoid(g[:, 16:32])
        gg = jnp.tanh(g[:, 32:48])
        o = jax.nn.sigmoid(g[:, 48:64])
        c = f * c + i * gg
        h = o * jnp.tanh(c)
    o_ref[...] = jnp.dot(jnp.maximum(h, 0.0), wout_ref[...],
                         preferred_element_type=jnp.float32) + bout_ref[...]


def _lstm_head(x2, wih_t, whh_t, bih, bhh, wout_t, bout):
    grid = N // _LSTM_BN
    return pl.pallas_call(
        _lstm_body,
        grid=(grid,),
        in_specs=[
            pl.BlockSpec((T, _LSTM_BN, D), lambda i: (0, i, 0)),
            pl.BlockSpec((D, 64), lambda i: (0, 0)),
            pl.BlockSpec((16, 64), lambda i: (0, 0)),
            pl.BlockSpec((1, 64), lambda i: (0, 0)),
            pl.BlockSpec((1, 64), lambda i: (0, 0)),
            pl.BlockSpec((16, 2), lambda i: (0, 0)),
            pl.BlockSpec((1, 2), lambda i: (0, 0)),
        ],
        out_specs=pl.BlockSpec((_LSTM_BN, 2), lambda i: (i, 0)),
        out_shape=jax.ShapeDtypeStruct((N, 2), jnp.float32),
    )(x2, wih_t, whh_t, bih, bhh, wout_t, bout)


def kernel(hyperedge_seq, epoch, usr_emb, W1, b1, W2, b2,
           W_ih, W_hh, b_ih, b_hh, W_out, b_out):
    del epoch
    idx1 = hyperedge_seq.reshape(T, 2, NS, NCH1, K1)
    idx3 = hyperedge_seq.reshape(T, 2, NS, NCH3, K3)

    y0 = _mm_aug(usr_emb, W1)
    x1, binv, dinv, _ = _stage1(y0, idx1, b1)
    w2pad = jnp.concatenate([W2, jnp.zeros((L, D), jnp.float32)], axis=0)
    z = _mm_wide(x1, w2pad)
    x2, _ = _stage3(z.reshape(T, N, D), idx3, b2, binv, dinv)

    logits = _lstm_head(
        x2.reshape(T, N, D),
        W_ih.T, W_hh.T,
        b_ih.reshape(1, 64), b_hh.reshape(1, 64),
        W_out.T, b_out.reshape(1, 2),
    )
    return logits


# speedup vs baseline: 17.2889x; 1.1146x over previous
import functools

import jax
import jax.numpy as jnp
from jax import lax
from jax.experimental import pallas as pl
from jax.experimental.pallas import tpu as pltpu
from jax.experimental.pallas import tpu_sc as plsc

N = 10000
T = 8
E = 160000
D = 32
DA = 48
L = 16
NS = 16
NC = 2
EPT = E // NS
K1 = 250
NCH1 = EPT // K1
K3 = 500
NCH3 = EPT // K3
RPT = N // NS
TPC = T // NC
_U = 25


def _fill_rows(ref, nrows, ncols, val):
    v = jnp.full((L,), val, jnp.float32)

    def body(j, _):
        r = j * _U
        for u in range(_U):
            for col in range(ncols // L):
                ref[r + u, col * L:(col + 1) * L] = v
        return 0

    lax.fori_loop(0, nrows // _U, body, 0)


def _pass(table_view, idxg_ref, idxs_ref, rows3, acc, nch, sem_g, sem_s):
    scat = {}
    gat = {0: pltpu.async_copy(table_view.at[idxg_ref.at[0]], rows3[0],
                               sem_g)}
    for ch in range(nch):
        b = ch % 3
        if ch + 1 < nch:
            if ch - 2 >= 0:
                scat[ch - 2].wait()
            gat[ch + 1] = pltpu.async_copy(
                table_view.at[idxg_ref.at[ch + 1]], rows3[(ch + 1) % 3],
                sem_g)
        gat[ch].wait()
        scat[ch] = pltpu.async_copy(rows3[b], acc.at[idxs_ref.at[ch]], sem_s,
                                    add=True)
    for ch in range(max(0, nch - 3), nch):
        scat[ch].wait()


def _stage1_body(y0_hbm, idx_hbm, b1_hbm,
                 x1_hbm, binv_hbm, dinv_hbm, m_hbm,
                 acc, idx_n2, idx_e2, rows_a, rows_b, rows_c, fsl, csl, bv,
                 sem_i, sem_g, sem_s):
    c = lax.axis_index("c")
    s = lax.axis_index("s")
    r0 = s * RPT
    rows3 = [rows_a, rows_b, rows_c]

    pltpu.sync_copy(b1_hbm, bv)

    for k4 in range(TPC):
        t = c * TPC + k4

        di = pltpu.async_copy(idx_hbm.at[t].at[0].at[s], idx_n2, sem_i)
        de = pltpu.async_copy(idx_hbm.at[t].at[1].at[s], idx_e2, sem_i)
        _fill_rows(fsl, RPT, DA, 0.0)
        pltpu.sync_copy(fsl, acc.at[pl.ds(r0, RPT)])
        di.wait()
        de.wait()
        plsc.subcore_barrier()

        _pass(y0_hbm, idx_n2, idx_e2, rows3, acc, NCH1, sem_g, sem_s)
        plsc.subcore_barrier()

        pltpu.sync_copy(acc.at[pl.ds(r0, RPT)], fsl)
        onev = jnp.full((L,), 1.0, jnp.float32)

        def s1(j, _):
            for u in range(_U):
                r = j * _U + u
                cv = fsl[r, D:DA]
                inv = jnp.where(cv > 0.0, 1.0 / cv, 0.0)
                csl[r] = inv
                fsl[r, 0:L] = fsl[r, 0:L] * inv
                fsl[r, L:D] = fsl[r, L:D] * inv
                fsl[r, D:DA] = onev
            return 0

        lax.fori_loop(0, RPT // _U, s1, 0)
        pltpu.sync_copy(fsl, m_hbm.at[t].at[pl.ds(r0, RPT)])
        pltpu.sync_copy(csl, binv_hbm.at[pl.ds(t * N + r0, RPT)])
        _fill_rows(fsl, RPT, DA, 0.0)
        pltpu.sync_copy(fsl, acc.at[pl.ds(r0, RPT)])
        plsc.subcore_barrier()

        _pass(m_hbm.at[t], idx_e2, idx_n2, rows3, acc, NCH1, sem_g, sem_s)
        plsc.subcore_barrier()

        pltpu.sync_copy(acc.at[pl.ds(r0, RPT)], fsl)
        b0 = bv[0:L]
        b1v = bv[L:D]
        zv = jnp.zeros((L,), jnp.float32)

        def s2(j, _):
            for u in range(_U):
                r = j * _U + u
                cv = fsl[r, D:DA]
                inv = jnp.where(cv > 0.0, 1.0 / cv, 0.0)
                csl[r] = inv
                fsl[r, 0:L] = jnp.maximum(fsl[r, 0:L] * inv + b0, 0.0)
                fsl[r, L:D] = jnp.maximum(fsl[r, L:D] * inv + b1v, 0.0)
                fsl[r, D:DA] = zv
            return 0

        lax.fori_loop(0, RPT // _U, s2, 0)
        pltpu.sync_copy(fsl, x1_hbm.at[pl.ds(t * N + r0, RPT)])
        pltpu.sync_copy(csl, dinv_hbm.at[pl.ds(t * N + r0, RPT)])


def _stage3_body(z_hbm, idx_hbm, b2_hbm, binv_hbm, dinv_hbm,
                 x2_hbm, m_hbm,
                 acc, idx_n2, idx_e2, rows_a, rows_b, rows_c, fsl, csl, bv,
                 sem_i, sem_g, sem_s):
    c = lax.axis_index("c")
    s = lax.axis_index("s")
    r0 = s * RPT
    rows3 = [rows_a, rows_b, rows_c]

    pltpu.sync_copy(b2_hbm, bv)

    for k4 in range(TPC):
        t = c * TPC + k4

        di = pltpu.async_copy(idx_hbm.at[t].at[0].at[s], idx_n2, sem_i)
        de = pltpu.async_copy(idx_hbm.at[t].at[1].at[s], idx_e2, sem_i)
        _fill_rows(fsl, RPT, D, 0.0)
        pltpu.sync_copy(fsl, acc.at[pl.ds(r0, RPT)])
        di.wait()
        de.wait()
        plsc.subcore_barrier()

        _pass(z_hbm.at[t], idx_n2, idx_e2, rows3, acc, NCH3, sem_g, sem_s)
        plsc.subcore_barrier()

        pltpu.sync_copy(acc.at[pl.ds(r0, RPT)], fsl)
        pltpu.sync_copy(binv_hbm.at[pl.ds(t * N + r0, RPT)], csl)

        def s1(j, _):
            for u in range(_U):
                r = j * _U + u
                inv = csl[r]
                fsl[r, 0:L] = fsl[r, 0:L] * inv
                fsl[r, L:D] = fsl[r, L:D] * inv
            return 0

        lax.fori_loop(0, RPT // _U, s1, 0)
        pltpu.sync_copy(fsl, m_hbm.at[t].at[pl.ds(r0, RPT)])
        _fill_rows(fsl, RPT, D, 0.0)
        pltpu.sync_copy(fsl, acc.at[pl.ds(r0, RPT)])
        plsc.subcore_barrier()

        _pass(m_hbm.at[t], idx_e2, idx_n2, rows3, acc, NCH3, sem_g, sem_s)
        plsc.subcore_barrier()

        pltpu.sync_copy(acc.at[pl.ds(r0, RPT)], fsl)
        pltpu.sync_copy(dinv_hbm.at[pl.ds(t * N + r0, RPT)], csl)
        b0 = bv[0:L]
        b1v = bv[L:D]

        def s2(j, _):
            for u in range(_U):
                r = j * _U + u
                inv = csl[r]
                fsl[r, 0:L] = jnp.maximum(fsl[r, 0:L] * inv + b0, 0.0)
                fsl[r, L:D] = jnp.maximum(fsl[r, L:D] * inv + b1v, 0.0)
            return 0

        lax.fori_loop(0, RPT // _U, s2, 0)
        pltpu.sync_copy(fsl, x2_hbm.at[pl.ds(t * N + r0, RPT)])


_SC_MESH = plsc.VectorSubcoreMesh(core_axis_name="c", subcore_axis_name="s")

_stage1 = pl.kernel(
    _stage1_body,
    out_type=[
        jax.ShapeDtypeStruct((T * N, DA), jnp.float32),
        jax.ShapeDtypeStruct((T * N, L), jnp.float32),
        jax.ShapeDtypeStruct((T * N, L), jnp.float32),
        jax.ShapeDtypeStruct((T, N, DA), jnp.float32),
    ],
    mesh=_SC_MESH,
    scratch_types=[
        pltpu.VMEM_SHARED((N, DA), jnp.float32),
        pltpu.VMEM((NCH1, K1), jnp.int32),
        pltpu.VMEM((NCH1, K1), jnp.int32),
        pltpu.VMEM((K1, DA), jnp.float32),
        pltpu.VMEM((K1, DA), jnp.float32),
        pltpu.VMEM((K1, DA), jnp.float32),
        pltpu.VMEM((RPT, DA), jnp.float32),
        pltpu.VMEM((RPT, L), jnp.float32),
        pltpu.VMEM((D,), jnp.float32),
        pltpu.SemaphoreType.DMA,
        pltpu.SemaphoreType.DMA,
        pltpu.SemaphoreType.DMA,
    ],
    compiler_params=pltpu.CompilerParams(use_tc_tiling_on_sc=False),
)

_stage3 = pl.kernel(
    _stage3_body,
    out_type=[
        jax.ShapeDtypeStruct((T * N, D), jnp.float32),
        jax.ShapeDtypeStruct((T, N, D), jnp.float32),
    ],
    mesh=_SC_MESH,
    scratch_types=[
        pltpu.VMEM_SHARED((N, D), jnp.float32),
        pltpu.VMEM((NCH3, K3), jnp.int32),
        pltpu.VMEM((NCH3, K3), jnp.int32),
        pltpu.VMEM((K3, D), jnp.float32),
        pltpu.VMEM((K3, D), jnp.float32),
        pltpu.VMEM((K3, D), jnp.float32),
        pltpu.VMEM((RPT, D), jnp.float32),
        pltpu.VMEM((RPT, L), jnp.float32),
        pltpu.VMEM((D,), jnp.float32),
        pltpu.SemaphoreType.DMA,
        pltpu.SemaphoreType.DMA,
        pltpu.SemaphoreType.DMA,
    ],
    compiler_params=pltpu.CompilerParams(use_tc_tiling_on_sc=False),
)


def _mm_aug_body(x_ref, w_ref, o_ref):
    xw = jnp.dot(x_ref[...], w_ref[...], preferred_element_type=jnp.float32)
    o_ref[...] = jnp.concatenate(
        [xw, jnp.ones((xw.shape[0], L), jnp.float32)], axis=1)


def _mm_aug(x, w):
    return pl.pallas_call(
        _mm_aug_body,
        grid=(1,),
        in_specs=[
            pl.BlockSpec((N, D), lambda i: (i, 0)),
            pl.BlockSpec((D, D), lambda i: (0, 0)),
        ],
        out_specs=pl.BlockSpec((N, DA), lambda i: (i, 0)),
        out_shape=jax.ShapeDtypeStruct((N, DA), jnp.float32),
    )(x, w)


def _mm_body(x_ref, w_ref, o_ref):
    o_ref[...] = jnp.dot(x_ref[...], w_ref[...],
                         preferred_element_type=jnp.float32)


def _mm_wide(x, w):
    m = x.shape[0]
    grid = m // N
    return pl.pallas_call(
        _mm_body,
        grid=(grid,),
        in_specs=[
            pl.BlockSpec((N, DA), lambda i: (i, 0)),
            pl.BlockSpec((DA, D), lambda i: (0, 0)),
        ],
        out_specs=pl.BlockSpec((N, D), lambda i: (i, 0)),
        out_shape=jax.ShapeDtypeStruct((m, D), jnp.float32),
    )(x, w)


_LSTM_BN = 2000


def _lstm_body(seq_ref, wih_ref, whh_ref, bih_ref, bhh_ref,
               wout_ref, bout_ref, o_ref):
    xall = seq_ref[...]
    wih = wih_ref[...]
    whh = whh_ref[...]
    b = bih_ref[...] + bhh_ref[...]
    h = jnp.zeros((_LSTM_BN, 16), jnp.float32)
    c = jnp.zeros((_LSTM_BN, 16), jnp.float32)
    for t in range(T):
        g = jnp.dot(xall[t], wih, preferred_element_type=jnp.float32) \
            + jnp.dot(h, whh, preferred_element_type=jnp.float32) + b
        i = jax.nn.sigmoid(g[:, 0:16])
        f = jax.nn.sigmoid(g[:, 16:32])
        gg = jnp.tanh(g[:, 32:48])
        o = jax.nn.sigmoid(g[:, 48:64])
        c = f * c + i * gg
        h = o * jnp.tanh(c)
    o_ref[...] = jnp.dot(jnp.maximum(h, 0.0), wout_ref[...],
                         preferred_element_type=jnp.float32) + bout_ref[...]


def _lstm_head(x2, wih_t, whh_t, bih, bhh, wout_t, bout):
    grid = N // _LSTM_BN
    return pl.pallas_call(
        _lstm_body,
        grid=(grid,),
        in_specs=[
            pl.BlockSpec((T, _LSTM_BN, D), lambda i: (0, i, 0)),
            pl.BlockSpec((D, 64), lambda i: (0, 0)),
            pl.BlockSpec((16, 64), lambda i: (0, 0)),
            pl.BlockSpec((1, 64), lambda i: (0, 0)),
            pl.BlockSpec((1, 64), lambda i: (0, 0)),
            pl.BlockSpec((16, 2), lambda i: (0, 0)),
            pl.BlockSpec((1, 2), lambda i: (0, 0)),
        ],
        out_specs=pl.BlockSpec((_LSTM_BN, 2), lambda i: (i, 0)),
        out_shape=jax.ShapeDtypeStruct((N, 2), jnp.float32),
    )(x2, wih_t, whh_t, bih, bhh, wout_t, bout)


def kernel(hyperedge_seq, epoch, usr_emb, W1, b1, W2, b2,
           W_ih, W_hh, b_ih, b_hh, W_out, b_out):
    del epoch
    idx1 = hyperedge_seq.reshape(T, 2, NS, NCH1, K1)
    idx3 = hyperedge_seq.reshape(T, 2, NS, NCH3, K3)

    y0 = _mm_aug(usr_emb, W1)
    x1, binv, dinv, _ = _stage1(y0, idx1, b1)
    w2pad = jnp.concatenate([W2, jnp.zeros((L, D), jnp.float32)], axis=0)
    z = _mm_wide(x1, w2pad)
    x2, _ = _stage3(z.reshape(T, N, D), idx3, b2, binv, dinv)

    logits = _lstm_head(
        x2.reshape(T, N, D),
        W_ih.T, W_hh.T,
        b_ih.reshape(1, 64), b_hh.reshape(1, 64),
        W_out.T, b_out.reshape(1, 2),
    )
    return logits


# speedup vs baseline: 18.1044x; 1.0472x over previous
import functools

import jax
import jax.numpy as jnp
from jax import lax
from jax.experimental import pallas as pl
from jax.experimental.pallas import tpu as pltpu
from jax.experimental.pallas import tpu_sc as plsc

N = 10000
T = 8
E = 160000
D = 32
L = 16
NS = 16
NC = 2
EPT = E // NS
K1 = 400
NCH1 = EPT // K1
K3 = 500
NCH3 = EPT // K3
RPT = N // NS
TPC = T // NC
_U = 25


def _fill_rows(ref, nrows, ncols, val):
    v = jnp.full((L,), val, jnp.float32)

    def body(j, _):
        r = j * _U
        for u in range(_U):
            for col in range(ncols // L):
                ref[r + u, col * L:(col + 1) * L] = v
        return 0

    lax.fori_loop(0, nrows // _U, body, 0)


def _pass(table_view, idxg_ref, idxs_ref, rows3, acc, nch, sem_g, sem_s,
          ones=None, accC=None, sem_c=None):
    scat = {}
    cnts = []
    gat = {0: pltpu.async_copy(table_view.at[idxg_ref.at[0]], rows3[0],
                               sem_g)}
    for ch in range(nch):
        b = ch % 3
        if ch + 1 < nch:
            if ch - 2 >= 0:
                scat[ch - 2].wait()
            gat[ch + 1] = pltpu.async_copy(
                table_view.at[idxg_ref.at[ch + 1]], rows3[(ch + 1) % 3],
                sem_g)
        if ones is not None:
            cnts.append(pltpu.async_copy(ones, accC.at[idxs_ref.at[ch]],
                                         sem_c, add=True))
        gat[ch].wait()
        scat[ch] = pltpu.async_copy(rows3[b], acc.at[idxs_ref.at[ch]], sem_s,
                                    add=True)
    for ch in range(max(0, nch - 3), nch):
        scat[ch].wait()
    for d in cnts:
        d.wait()


def _stage1_body(y0_hbm, idx_hbm, b1_hbm,
                 x1_hbm, binv_hbm, dinv_hbm, m_hbm,
                 accF, accC, idx_n2, idx_e2, rows_a, rows_b, rows_c, ones,
                 fsl, csl, bv,
                 sem_i, sem_g, sem_s, sem_c):
    c = lax.axis_index("c")
    s = lax.axis_index("s")
    r0 = s * RPT
    rows3 = [rows_a, rows_b, rows_c]

    pltpu.sync_copy(b1_hbm, bv)
    _fill_rows(ones, K1, L, 1.0)

    for k4 in range(TPC):
        t = c * TPC + k4

        di = pltpu.async_copy(idx_hbm.at[t].at[0].at[s], idx_n2, sem_i)
        de = pltpu.async_copy(idx_hbm.at[t].at[1].at[s], idx_e2, sem_i)
        _fill_rows(fsl, RPT, D, 0.0)
        pltpu.sync_copy(fsl, accF.at[pl.ds(r0, RPT)])
        _fill_rows(csl, RPT, L, 0.0)
        pltpu.sync_copy(csl, accC.at[pl.ds(r0, RPT)])
        di.wait()
        de.wait()
        plsc.subcore_barrier()

        _pass(y0_hbm, idx_n2, idx_e2, rows3, accF, NCH1, sem_g, sem_s,
              ones=ones, accC=accC, sem_c=sem_c)
        plsc.subcore_barrier()

        pltpu.sync_copy(accF.at[pl.ds(r0, RPT)], fsl)
        pltpu.sync_copy(accC.at[pl.ds(r0, RPT)], csl)

        def s1(j, _):
            for u in range(_U):
                r = j * _U + u
                cv = csl[r]
                inv = jnp.where(cv > 0.0, 1.0 / cv, 0.0)
                csl[r] = inv
                fsl[r, 0:L] = fsl[r, 0:L] * inv
                fsl[r, L:D] = fsl[r, L:D] * inv
            return 0

        lax.fori_loop(0, RPT // _U, s1, 0)
        pltpu.sync_copy(fsl, m_hbm.at[t].at[pl.ds(r0, RPT)])
        pltpu.sync_copy(csl, binv_hbm.at[pl.ds(t * N + r0, RPT)])
        _fill_rows(fsl, RPT, D, 0.0)
        pltpu.sync_copy(fsl, accF.at[pl.ds(r0, RPT)])
        _fill_rows(csl, RPT, L, 0.0)
        pltpu.sync_copy(csl, accC.at[pl.ds(r0, RPT)])
        plsc.subcore_barrier()

        _pass(m_hbm.at[t], idx_e2, idx_n2, rows3, accF, NCH1, sem_g, sem_s,
              ones=ones, accC=accC, sem_c=sem_c)
        plsc.subcore_barrier()

        pltpu.sync_copy(accF.at[pl.ds(r0, RPT)], fsl)
        pltpu.sync_copy(accC.at[pl.ds(r0, RPT)], csl)
        b0 = bv[0:L]
        b1v = bv[L:D]

        def s2(j, _):
            for u in range(_U):
                r = j * _U + u
                cv = csl[r]
                inv = jnp.where(cv > 0.0, 1.0 / cv, 0.0)
                csl[r] = inv
                fsl[r, 0:L] = jnp.maximum(fsl[r, 0:L] * inv + b0, 0.0)
                fsl[r, L:D] = jnp.maximum(fsl[r, L:D] * inv + b1v, 0.0)
            return 0

        lax.fori_loop(0, RPT // _U, s2, 0)
        pltpu.sync_copy(fsl, x1_hbm.at[pl.ds(t * N + r0, RPT)])
        pltpu.sync_copy(csl, dinv_hbm.at[pl.ds(t * N + r0, RPT)])


def _stage3_body(z_hbm, idx_hbm, b2_hbm, binv_hbm, dinv_hbm,
                 x2_hbm, m_hbm,
                 acc, idx_n2, idx_e2, rows_a, rows_b, rows_c, fsl, csl, bv,
                 sem_i, sem_g, sem_s):
    c = lax.axis_index("c")
    s = lax.axis_index("s")
    r0 = s * RPT
    rows3 = [rows_a, rows_b, rows_c]

    pltpu.sync_copy(b2_hbm, bv)

    for k4 in range(TPC):
        t = c * TPC + k4

        di = pltpu.async_copy(idx_hbm.at[t].at[0].at[s], idx_n2, sem_i)
        de = pltpu.async_copy(idx_hbm.at[t].at[1].at[s], idx_e2, sem_i)
        _fill_rows(fsl, RPT, D, 0.0)
        pltpu.sync_copy(fsl, acc.at[pl.ds(r0, RPT)])
        di.wait()
        de.wait()
        plsc.subcore_barrier()

        _pass(z_hbm.at[t], idx_n2, idx_e2, rows3, acc, NCH3, sem_g, sem_s)
        plsc.subcore_barrier()

        pltpu.sync_copy(acc.at[pl.ds(r0, RPT)], fsl)
        pltpu.sync_copy(binv_hbm.at[pl.ds(t * N + r0, RPT)], csl)

        def s1(j, _):
            for u in range(_U):
                r = j * _U + u
                inv = csl[r]
                fsl[r, 0:L] = fsl[r, 0:L] * inv
                fsl[r, L:D] = fsl[r, L:D] * inv
            return 0

        lax.fori_loop(0, RPT // _U, s1, 0)
        pltpu.sync_copy(fsl, m_hbm.at[t].at[pl.ds(r0, RPT)])
        _fill_rows(fsl, RPT, D, 0.0)
        pltpu.sync_copy(fsl, acc.at[pl.ds(r0, RPT)])
        plsc.subcore_barrier()

        _pass(m_hbm.at[t], idx_e2, idx_n2, rows3, acc, NCH3, sem_g, sem_s)
        plsc.subcore_barrier()

        pltpu.sync_copy(acc.at[pl.ds(r0, RPT)], fsl)
        pltpu.sync_copy(dinv_hbm.at[pl.ds(t * N + r0, RPT)], csl)
        b0 = bv[0:L]
        b1v = bv[L:D]

        def s2(j, _):
            for u in range(_U):
                r = j * _U + u
                inv = csl[r]
                fsl[r, 0:L] = jnp.maximum(fsl[r, 0:L] * inv + b0, 0.0)
                fsl[r, L:D] = jnp.maximum(fsl[r, L:D] * inv + b1v, 0.0)
            return 0

        lax.fori_loop(0, RPT // _U, s2, 0)
        pltpu.sync_copy(fsl, x2_hbm.at[pl.ds(t * N + r0, RPT)])


_SC_MESH = plsc.VectorSubcoreMesh(core_axis_name="c", subcore_axis_name="s")

_stage1 = pl.kernel(
    _stage1_body,
    out_type=[
        jax.ShapeDtypeStruct((T * N, D), jnp.float32),
        jax.ShapeDtypeStruct((T * N, L), jnp.float32),
        jax.ShapeDtypeStruct((T * N, L), jnp.float32),
        jax.ShapeDtypeStruct((T, N, D), jnp.float32),
    ],
    mesh=_SC_MESH,
    scratch_types=[
        pltpu.VMEM_SHARED((N, D), jnp.float32),
        pltpu.VMEM_SHARED((N, L), jnp.float32),
        pltpu.VMEM((NCH1, K1), jnp.int32),
        pltpu.VMEM((NCH1, K1), jnp.int32),
        pltpu.VMEM((K1, D), jnp.float32),
        pltpu.VMEM((K1, D), jnp.float32),
        pltpu.VMEM((K1, D), jnp.float32),
        pltpu.VMEM((K1, L), jnp.float32),
        pltpu.VMEM((RPT, D), jnp.float32),
        pltpu.VMEM((RPT, L), jnp.float32),
        pltpu.VMEM((D,), jnp.float32),
        pltpu.SemaphoreType.DMA,
        pltpu.SemaphoreType.DMA,
        pltpu.SemaphoreType.DMA,
        pltpu.SemaphoreType.DMA,
    ],
    compiler_params=pltpu.CompilerParams(use_tc_tiling_on_sc=False),
)

_stage3 = pl.kernel(
    _stage3_body,
    out_type=[
        jax.ShapeDtypeStruct((T * N, D), jnp.float32),
        jax.ShapeDtypeStruct((T, N, D), jnp.float32),
    ],
    mesh=_SC_MESH,
    scratch_types=[
        pltpu.VMEM_SHARED((N, D), jnp.float32),
        pltpu.VMEM((NCH3, K3), jnp.int32),
        pltpu.VMEM((NCH3, K3), jnp.int32),
        pltpu.VMEM((K3, D), jnp.float32),
        pltpu.VMEM((K3, D), jnp.float32),
        pltpu.VMEM((K3, D), jnp.float32),
        pltpu.VMEM((RPT, D), jnp.float32),
        pltpu.VMEM((RPT, L), jnp.float32),
        pltpu.VMEM((D,), jnp.float32),
        pltpu.SemaphoreType.DMA,
        pltpu.SemaphoreType.DMA,
        pltpu.SemaphoreType.DMA,
    ],
    compiler_params=pltpu.CompilerParams(use_tc_tiling_on_sc=False),
)


def _mm_body(x_ref, w_ref, o_ref):
    o_ref[...] = jnp.dot(x_ref[...], w_ref[...],
                         preferred_element_type=jnp.float32)


def _matmul(x, w):
    m = x.shape[0]
    grid = m // N
    return pl.pallas_call(
        _mm_body,
        grid=(grid,),
        in_specs=[
            pl.BlockSpec((N, D), lambda i: (i, 0)),
            pl.BlockSpec((D, D), lambda i: (0, 0)),
        ],
        out_specs=pl.BlockSpec((N, D), lambda i: (i, 0)),
        out_shape=jax.ShapeDtypeStruct((m, D), jnp.float32),
    )(x, w)


_LSTM_BN = 2000


def _lstm_body(seq_ref, wih_ref, whh_ref, bih_ref, bhh_ref,
               wout_ref, bout_ref, o_ref):
    xall = seq_ref[...]
    wih = wih_ref[...]
    whh = whh_ref[...]
    b = bih_ref[...] + bhh_ref[...]
    h = jnp.zeros((_LSTM_BN, 16), jnp.float32)
    c = jnp.zeros((_LSTM_BN, 16), jnp.float32)
    for t in range(T):
        g = jnp.dot(xall[t], wih, preferred_element_type=jnp.float32) \
            + jnp.dot(h, whh, preferred_element_type=jnp.float32) + b
        i = jax.nn.sigmoid(g[:, 0:16])
        f = jax.nn.sigmoid(g[:, 16:32])
        gg = jnp.tanh(g[:, 32:48])
        o = jax.nn.sigmoid(g[:, 48:64])
        c = f * c + i * gg
        h = o * jnp.tanh(c)
    o_ref[...] = jnp.dot(jnp.maximum(h, 0.0), wout_ref[...],
                         preferred_element_type=jnp.float32) + bout_ref[...]


def _lstm_head(x2, wih_t, whh_t, bih, bhh, wout_t, bout):
    grid = N // _LSTM_BN
    return pl.pallas_call(
        _lstm_body,
        grid=(grid,),
        in_specs=[
            pl.BlockSpec((T, _LSTM_BN, D), lambda i: (0, i, 0)),
            pl.BlockSpec((D, 64), lambda i: (0, 0)),
            pl.BlockSpec((16, 64), lambda i: (0, 0)),
            pl.BlockSpec((1, 64), lambda i: (0, 0)),
            pl.BlockSpec((1, 64), lambda i: (0, 0)),
            pl.BlockSpec((16, 2), lambda i: (0, 0)),
            pl.BlockSpec((1, 2), lambda i: (0, 0)),
        ],
        out_specs=pl.BlockSpec((_LSTM_BN, 2), lambda i: (i, 0)),
        out_shape=jax.ShapeDtypeStruct((N, 2), jnp.float32),
    )(x2, wih_t, whh_t, bih, bhh, wout_t, bout)


def kernel(hyperedge_seq, epoch, usr_emb, W1, b1, W2, b2,
           W_ih, W_hh, b_ih, b_hh, W_out, b_out):
    del epoch
    idx1 = hyperedge_seq.reshape(T, 2, NS, NCH1, K1)
    idx3 = hyperedge_seq.reshape(T, 2, NS, NCH3, K3)

    y0 = _matmul(usr_emb, W1)
    x1, binv, dinv, _ = _stage1(y0, idx1, b1)
    z = _matmul(x1, W2)
    x2, _ = _stage3(z.reshape(T, N, D), idx3, b2, binv, dinv)

    logits = _lstm_head(
        x2.reshape(T, N, D),
        W_ih.T, W_hh.T,
        b_ih.reshape(1, 64), b_hh.reshape(1, 64),
        W_out.T, b_out.reshape(1, 2),
    )
    return logits


# speedup vs baseline: 19.1579x; 1.0582x over previous
import functools

import jax
import jax.numpy as jnp
from jax import lax
from jax.experimental import pallas as pl
from jax.experimental.pallas import tpu as pltpu
from jax.experimental.pallas import tpu_sc as plsc

N = 10000
T = 8
E = 160000
D = 32
L = 16
NS = 16
NC = 2
EPT = E // NS
K1 = 400
NCH1 = EPT // K1
K3 = 400
NCH3 = EPT // K3
RPT = N // NS
TPC = T // NC
_U = 25


def _fill_rows(ref, nrows, ncols, val):
    v = jnp.full((L,), val, jnp.float32)

    def body(j, _):
        r = j * _U
        for u in range(_U):
            for col in range(ncols // L):
                ref[r + u, col * L:(col + 1) * L] = v
        return 0

    lax.fori_loop(0, nrows // _U, body, 0)


def _pass(table_view, idxg_ref, idxs_ref, rows3, acc, nch, sem_g, sem_s,
          ones=None, accC=None, sem_c=None):
    scat = {}
    cnts = []
    gat = {0: pltpu.async_copy(table_view.at[idxg_ref.at[0]], rows3[0],
                               sem_g)}
    for ch in range(nch):
        b = ch % 3
        if ch + 1 < nch:
            if ch - 2 >= 0:
                scat[ch - 2].wait()
            gat[ch + 1] = pltpu.async_copy(
                table_view.at[idxg_ref.at[ch + 1]], rows3[(ch + 1) % 3],
                sem_g)
        if ones is not None:
            cnts.append(pltpu.async_copy(ones, accC.at[idxs_ref.at[ch]],
                                         sem_c, add=True))
        gat[ch].wait()
        scat[ch] = pltpu.async_copy(rows3[b], acc.at[idxs_ref.at[ch]], sem_s,
                                    add=True)
    for ch in range(max(0, nch - 3), nch):
        scat[ch].wait()
    for d in cnts:
        d.wait()


def _stage1_body(y0_hbm, idx_hbm, b1_hbm,
                 x1_hbm, binv_hbm, dinv_hbm, m_hbm,
                 accF, accC, idx_n2, idx_e2, rows_a, rows_b, rows_c, ones,
                 fsl, csl, bv,
                 sem_i, sem_g, sem_s, sem_c):
    c = lax.axis_index("c")
    s = lax.axis_index("s")
    r0 = s * RPT
    rows3 = [rows_a, rows_b, rows_c]

    pltpu.sync_copy(b1_hbm, bv)
    _fill_rows(ones, K1, L, 1.0)

    for k4 in range(TPC):
        t = c * TPC + k4

        di = pltpu.async_copy(idx_hbm.at[t].at[0].at[s], idx_n2, sem_i)
        de = pltpu.async_copy(idx_hbm.at[t].at[1].at[s], idx_e2, sem_i)
        _fill_rows(fsl, RPT, D, 0.0)
        pltpu.sync_copy(fsl, accF.at[pl.ds(r0, RPT)])
        _fill_rows(csl, RPT, L, 0.0)
        pltpu.sync_copy(csl, accC.at[pl.ds(r0, RPT)])
        di.wait()
        de.wait()
        plsc.subcore_barrier()

        _pass(y0_hbm, idx_n2, idx_e2, rows3, accF, NCH1, sem_g, sem_s,
              ones=ones, accC=accC, sem_c=sem_c)
        plsc.subcore_barrier()

        pltpu.sync_copy(accF.at[pl.ds(r0, RPT)], fsl)
        pltpu.sync_copy(accC.at[pl.ds(r0, RPT)], csl)

        def s1(j, _):
            for u in range(_U):
                r = j * _U + u
                cv = csl[r]
                inv = jnp.where(cv > 0.0, 1.0 / cv, 0.0)
                csl[r] = inv
                fsl[r, 0:L] = fsl[r, 0:L] * inv
                fsl[r, L:D] = fsl[r, L:D] * inv
            return 0

        lax.fori_loop(0, RPT // _U, s1, 0)
        pltpu.sync_copy(fsl, m_hbm.at[t].at[pl.ds(r0, RPT)])
        pltpu.sync_copy(csl, binv_hbm.at[pl.ds(t * N + r0, RPT)])
        _fill_rows(fsl, RPT, D, 0.0)
        pltpu.sync_copy(fsl, accF.at[pl.ds(r0, RPT)])
        _fill_rows(csl, RPT, L, 0.0)
        pltpu.sync_copy(csl, accC.at[pl.ds(r0, RPT)])
        plsc.subcore_barrier()

        _pass(m_hbm.at[t], idx_e2, idx_n2, rows3, accF, NCH1, sem_g, sem_s,
              ones=ones, accC=accC, sem_c=sem_c)
        plsc.subcore_barrier()

        pltpu.sync_copy(accF.at[pl.ds(r0, RPT)], fsl)
        pltpu.sync_copy(accC.at[pl.ds(r0, RPT)], csl)
        b0 = bv[0:L]
        b1v = bv[L:D]

        def s2(j, _):
            for u in range(_U):
                r = j * _U + u
                cv = csl[r]
                inv = jnp.where(cv > 0.0, 1.0 / cv, 0.0)
                csl[r] = inv
                fsl[r, 0:L] = jnp.maximum(fsl[r, 0:L] * inv + b0, 0.0)
                fsl[r, L:D] = jnp.maximum(fsl[r, L:D] * inv + b1v, 0.0)
            return 0

        lax.fori_loop(0, RPT // _U, s2, 0)
        pltpu.sync_copy(fsl, x1_hbm.at[pl.ds(t * N + r0, RPT)])
        pltpu.sync_copy(csl, dinv_hbm.at[pl.ds(t * N + r0, RPT)])


def _stage3_body(z_hbm, idx_hbm, b2_hbm, binv_hbm, dinv_hbm,
                 x2_hbm,
                 acc, m_sp, idx_n2, idx_e2, rows_a, rows_b, rows_c, fsl, csl,
                 bv, sem_i, sem_g, sem_s):
    c = lax.axis_index("c")
    s = lax.axis_index("s")
    r0 = s * RPT
    rows3 = [rows_a, rows_b, rows_c]

    pltpu.sync_copy(b2_hbm, bv)

    for k4 in range(TPC):
        t = c * TPC + k4

        di = pltpu.async_copy(idx_hbm.at[t].at[0].at[s], idx_n2, sem_i)
        de = pltpu.async_copy(idx_hbm.at[t].at[1].at[s], idx_e2, sem_i)
        _fill_rows(fsl, RPT, D, 0.0)
        pltpu.sync_copy(fsl, acc.at[pl.ds(r0, RPT)])
        di.wait()
        de.wait()
        plsc.subcore_barrier()

        _pass(z_hbm.at[t], idx_n2, idx_e2, rows3, acc, NCH3, sem_g, sem_s)
        plsc.subcore_barrier()

        pltpu.sync_copy(acc.at[pl.ds(r0, RPT)], fsl)
        pltpu.sync_copy(binv_hbm.at[pl.ds(t * N + r0, RPT)], csl)

        def s1(j, _):
            for u in range(_U):
                r = j * _U + u
                inv = csl[r]
                fsl[r, 0:L] = fsl[r, 0:L] * inv
                fsl[r, L:D] = fsl[r, L:D] * inv
            return 0

        lax.fori_loop(0, RPT // _U, s1, 0)
        pltpu.sync_copy(fsl, m_sp.at[pl.ds(r0, RPT)])
        _fill_rows(fsl, RPT, D, 0.0)
        pltpu.sync_copy(fsl, acc.at[pl.ds(r0, RPT)])
        plsc.subcore_barrier()

        _pass(m_sp, idx_e2, idx_n2, rows3, acc, NCH3, sem_g, sem_s)
        plsc.subcore_barrier()

        pltpu.sync_copy(acc.at[pl.ds(r0, RPT)], fsl)
        pltpu.sync_copy(dinv_hbm.at[pl.ds(t * N + r0, RPT)], csl)
        b0 = bv[0:L]
        b1v = bv[L:D]

        def s2(j, _):
            for u in range(_U):
                r = j * _U + u
                inv = csl[r]
                fsl[r, 0:L] = jnp.maximum(fsl[r, 0:L] * inv + b0, 0.0)
                fsl[r, L:D] = jnp.maximum(fsl[r, L:D] * inv + b1v, 0.0)
            return 0

        lax.fori_loop(0, RPT // _U, s2, 0)
        pltpu.sync_copy(fsl, x2_hbm.at[pl.ds(t * N + r0, RPT)])


_SC_MESH = plsc.VectorSubcoreMesh(core_axis_name="c", subcore_axis_name="s")

_stage1 = pl.kernel(
    _stage1_body,
    out_type=[
        jax.ShapeDtypeStruct((T * N, D), jnp.float32),
        jax.ShapeDtypeStruct((T * N, L), jnp.float32),
        jax.ShapeDtypeStruct((T * N, L), jnp.float32),
        jax.ShapeDtypeStruct((T, N, D), jnp.float32),
    ],
    mesh=_SC_MESH,
    scratch_types=[
        pltpu.VMEM_SHARED((N, D), jnp.float32),
        pltpu.VMEM_SHARED((N, L), jnp.float32),
        pltpu.VMEM((NCH1, K1), jnp.int32),
        pltpu.VMEM((NCH1, K1), jnp.int32),
        pltpu.VMEM((K1, D), jnp.float32),
        pltpu.VMEM((K1, D), jnp.float32),
        pltpu.VMEM((K1, D), jnp.float32),
        pltpu.VMEM((K1, L), jnp.float32),
        pltpu.VMEM((RPT, D), jnp.float32),
        pltpu.VMEM((RPT, L), jnp.float32),
        pltpu.VMEM((D,), jnp.float32),
        pltpu.SemaphoreType.DMA,
        pltpu.SemaphoreType.DMA,
        pltpu.SemaphoreType.DMA,
        pltpu.SemaphoreType.DMA,
    ],
    compiler_params=pltpu.CompilerParams(use_tc_tiling_on_sc=False),
)

_stage3 = pl.kernel(
    _stage3_body,
    out_type=jax.ShapeDtypeStruct((T * N, D), jnp.float32),
    mesh=_SC_MESH,
    scratch_types=[
        pltpu.VMEM_SHARED((N, D), jnp.float32),
        pltpu.VMEM_SHARED((N, D), jnp.float32),
        pltpu.VMEM((NCH3, K3), jnp.int32),
        pltpu.VMEM((NCH3, K3), jnp.int32),
        pltpu.VMEM((K3, D), jnp.float32),
        pltpu.VMEM((K3, D), jnp.float32),
        pltpu.VMEM((K3, D), jnp.float32),
        pltpu.VMEM((RPT, D), jnp.float32),
        pltpu.VMEM((RPT, L), jnp.float32),
        pltpu.VMEM((D,), jnp.float32),
        pltpu.SemaphoreType.DMA,
        pltpu.SemaphoreType.DMA,
        pltpu.SemaphoreType.DMA,
    ],
    compiler_params=pltpu.CompilerParams(use_tc_tiling_on_sc=False),
)


def _mm_body(x_ref, w_ref, o_ref):
    o_ref[...] = jnp.dot(x_ref[...], w_ref[...],
                         preferred_element_type=jnp.float32)


def _matmul(x, w):
    m = x.shape[0]
    grid = m // N
    return pl.pallas_call(
        _mm_body,
        grid=(grid,),
        in_specs=[
            pl.BlockSpec((N, D), lambda i: (i, 0)),
            pl.BlockSpec((D, D), lambda i: (0, 0)),
        ],
        out_specs=pl.BlockSpec((N, D), lambda i: (i, 0)),
        out_shape=jax.ShapeDtypeStruct((m, D), jnp.float32),
    )(x, w)


_LSTM_BN = 2000


def _lstm_body(seq_ref, wih_ref, whh_ref, bih_ref, bhh_ref,
               wout_ref, bout_ref, o_ref):
    xall = seq_ref[...]
    wih = wih_ref[...]
    whh = whh_ref[...]
    b = bih_ref[...] + bhh_ref[...]
    h = jnp.zeros((_LSTM_BN, 16), jnp.float32)
    c = jnp.zeros((_LSTM_BN, 16), jnp.float32)
    for t in range(T):
        g = jnp.dot(xall[t], wih, preferred_element_type=jnp.float32) \
            + jnp.dot(h, whh, preferred_element_type=jnp.float32) + b
        i = jax.nn.sigmoid(g[:, 0:16])
        f = jax.nn.sigmoid(g[:, 16:32])
        gg = jnp.tanh(g[:, 32:48])
        o = jax.nn.sigmoid(g[:, 48:64])
        c = f * c + i * gg
        h = o * jnp.tanh(c)
    o_ref[...] = jnp.dot(jnp.maximum(h, 0.0), wout_ref[...],
                         preferred_element_type=jnp.float32) + bout_ref[...]


def _lstm_head(x2, wih_t, whh_t, bih, bhh, wout_t, bout):
    grid = N // _LSTM_BN
    return pl.pallas_call(
        _lstm_body,
        grid=(grid,),
        in_specs=[
            pl.BlockSpec((T, _LSTM_BN, D), lambda i: (0, i, 0)),
            pl.BlockSpec((D, 64), lambda i: (0, 0)),
            pl.BlockSpec((16, 64), lambda i: (0, 0)),
            pl.BlockSpec((1, 64), lambda i: (0, 0)),
            pl.BlockSpec((1, 64), lambda i: (0, 0)),
            pl.BlockSpec((16, 2), lambda i: (0, 0)),
            pl.BlockSpec((1, 2), lambda i: (0, 0)),
        ],
        out_specs=pl.BlockSpec((_LSTM_BN, 2), lambda i: (i, 0)),
        out_shape=jax.ShapeDtypeStruct((N, 2), jnp.float32),
    )(x2, wih_t, whh_t, bih, bhh, wout_t, bout)


def kernel(hyperedge_seq, epoch, usr_emb, W1, b1, W2, b2,
           W_ih, W_hh, b_ih, b_hh, W_out, b_out):
    del epoch
    idx1 = hyperedge_seq.reshape(T, 2, NS, NCH1, K1)
    idx3 = hyperedge_seq.reshape(T, 2, NS, NCH3, K3)

    y0 = _matmul(usr_emb, W1)
    x1, binv, dinv, _ = _stage1(y0, idx1, b1)
    z = _matmul(x1, W2)
    x2 = _stage3(z.reshape(T, N, D), idx3, b2, binv, dinv)

    logits = _lstm_head(
        x2.reshape(T, N, D),
        W_ih.T, W_hh.T,
        b_ih.reshape(1, 64), b_hh.reshape(1, 64),
        W_out.T, b_out.reshape(1, 2),
    )
    return logits
